# Initial kernel scaffold; baseline (speedup 1.0000x reference)
import jax
import jax.numpy as jnp
from jax.experimental import pallas as pl


def kernel(x, edge_index, batch, W0, b0, W1, b1, W2, b2, g0, be0, g1, be1, g2, be2, A1w, A1b, A2w, A2b, P1w, P1b, P2w, P2b):
    raise NotImplementedError("write your pallas kernel here")


# speedup vs baseline: 7.8428x; 7.8428x over previous
import functools

import jax
import jax.numpy as jnp
from jax import lax
from jax.experimental import pallas as pl
from jax.experimental.pallas import tpu as pltpu
from jax.experimental.pallas import tpu_sc as plsc

N = 10000
H = 128
G = 64
NC = 2
NS = 16
NW = NC * NS
LANES = 16
CHUNK = 128
SG = 8
NPAD = 10112
ROWS_PER_TILE = NPAD // NS
ZB = CHUNK

_MESH = plsc.VectorSubcoreMesh(
    core_axis_name="c", subcore_axis_name="s", num_cores=NC, num_subcores=NS)


def _num_chunks(num_edges):
  c = -(-num_edges // (NW * CHUNK))
  return -(-c // SG) * SG


def _deg_body(chunks, dst_hbm, out_hbm, didx, degv):
  wid = lax.axis_index("s") * NC + lax.axis_index("c")
  pltpu.sync_copy(dst_hbm.at[wid], didx)

  zeros16 = jnp.zeros((LANES,), jnp.float32)

  def zero_body(i, _):
    degv[pl.ds(i * LANES, LANES)] = zeros16
    return 0

  lax.fori_loop(0, NPAD // LANES, zero_body, 0)

  ones16 = jnp.ones((LANES,), jnp.float32)

  def edge_body(j, _):
    for k in range(CHUNK // LANES):
      idx = didx[j, pl.ds(k * LANES, LANES)]
      plsc.addupdate_scatter(degv, [idx], ones16)
    return 0

  lax.fori_loop(0, chunks, edge_body, 0)
  pltpu.sync_copy(degv, out_hbm.at[wid])


def _deg_call(dst3):
  chunks = dst3.shape[1]
  fn = pl.kernel(
      functools.partial(_deg_body, chunks),
      out_type=jax.ShapeDtypeStruct((NW, NPAD), jnp.float32),
      mesh=_MESH,
      compiler_params=pltpu.CompilerParams(needs_layout_passes=False),
      scratch_types=[
          pltpu.VMEM((chunks, CHUNK), jnp.int32),
          pltpu.VMEM((NPAD,), jnp.float32),
      ],
  )
  return fn(dst3)


def _agg_body(chunks, zh_hbm, src_hbm, dst_hbm, out_hbm,
              sidx, didx, msg0, msg1, acc, sem0, sem1):
  cid = lax.axis_index("c")
  sid = lax.axis_index("s")
  wid = sid * NC + cid

  zeros16 = jnp.zeros((LANES,), jnp.float32)

  def zb_body(i, _):
    for c in range(H // LANES):
      msg0[i, pl.ds(c * LANES, LANES)] = zeros16
    return 0

  lax.fori_loop(0, ZB, zb_body, 0)

  base = sid * ROWS_PER_TILE
  nfull = ROWS_PER_TILE // ZB
  rem = ROWS_PER_TILE % ZB
  for k in range(nfull):
    pltpu.sync_copy(msg0, acc.at[pl.ds(base + k * ZB, ZB)])
  if rem:
    pltpu.sync_copy(msg0.at[pl.ds(0, rem)],
                    acc.at[pl.ds(base + nfull * ZB, rem)])
  plsc.subcore_barrier()

  def group_body(q, _):
    pltpu.sync_copy(src_hbm.at[wid, pl.ds(q * SG, SG)], sidx)
    pltpu.sync_copy(dst_hbm.at[wid, pl.ds(q * SG, SG)], didx)

    def pair_body(t, _):
      a = 2 * t
      b = 2 * t + 1
      ca = pltpu.async_copy(zh_hbm.at[sidx.at[a]], msg0, sem0)
      cb = pltpu.async_copy(zh_hbm.at[sidx.at[b]], msg1, sem1)
      ca.wait()
      pltpu.sync_copy(msg0, acc.at[didx.at[a]], add=True)
      cb.wait()
      pltpu.sync_copy(msg1, acc.at[didx.at[b]], add=True)
      return 0

    lax.fori_loop(0, SG // 2, pair_body, 0)
    return 0

  lax.fori_loop(0, chunks // SG, group_body, 0)
  plsc.subcore_barrier()
  pltpu.sync_copy(acc.at[pl.ds(base, ROWS_PER_TILE)],
                  out_hbm.at[cid, pl.ds(base, ROWS_PER_TILE)])


def _agg_call(zh, src3, dst3):
  chunks = src3.shape[1]
  fn = pl.kernel(
      functools.partial(_agg_body, chunks),
      out_type=jax.ShapeDtypeStruct((NC, NPAD, H), jnp.float32),
      mesh=_MESH,
      compiler_params=pltpu.CompilerParams(needs_layout_passes=False),
      scratch_types=[
          pltpu.VMEM((SG, CHUNK), jnp.int32),
          pltpu.VMEM((SG, CHUNK), jnp.int32),
          pltpu.VMEM((CHUNK, H), jnp.float32),
          pltpu.VMEM((CHUNK, H), jnp.float32),
          pltpu.VMEM_SHARED((NPAD, H), jnp.float32),
          pltpu.SemaphoreType.DMA,
          pltpu.SemaphoreType.DMA,
      ],
  )
  return fn(zh, src3, dst3)


def _tc_first_body(degt_ref, x_ref, w_ref, zh_ref, dinv_ref):
  deg = jnp.sum(degt_ref[:, :], axis=1, keepdims=True) + 1.0
  dinv = lax.rsqrt(deg)[:N]
  z = jnp.dot(x_ref[:, :], w_ref[:, :], preferred_element_type=jnp.float32)
  zh_ref[:, :] = dinv * z
  dinv_ref[:, :] = dinv


def _tc_first(degt, x, w0):
  return pl.pallas_call(
      _tc_first_body,
      out_shape=[
          jax.ShapeDtypeStruct((N, H), jnp.float32),
          jax.ShapeDtypeStruct((N, 1), jnp.float32),
      ],
  )(degt, x, w0)


def _layer_math(accp_ref, zh_ref, dinv_ref, b_ref, g_ref, be_ref):
  acc = accp_ref[0, :N, :] + accp_ref[1, :N, :]
  dinv = dinv_ref[:, :]
  u = dinv * (acc + zh_ref[:, :]) + b_ref[:, :]
  m = jnp.mean(u, axis=0, keepdims=True)
  v = jnp.mean((u - m) * (u - m), axis=0, keepdims=True)
  y = g_ref[:, :] * (u - m) / jnp.sqrt(v + 1e-5) + be_ref[:, :]
  return jnp.maximum(y, 0.0), dinv


def _tc_mid_body(residual, accp_ref, zh_ref, dinv_ref, hprev_ref,
                 b_ref, g_ref, be_ref, w_ref, h_ref, zhn_ref):
  y, dinv = _layer_math(accp_ref, zh_ref, dinv_ref, b_ref, g_ref, be_ref)
  if residual:
    y = y + hprev_ref[:, :]
  h_ref[:, :] = y
  zhn_ref[:, :] = dinv * jnp.dot(
      y, w_ref[:, :], preferred_element_type=jnp.float32)


def _tc_mid(residual, accp, zh, dinv, hprev, b, g, be, w):
  return pl.pallas_call(
      functools.partial(_tc_mid_body, residual),
      out_shape=[
          jax.ShapeDtypeStruct((N, H), jnp.float32),
          jax.ShapeDtypeStruct((N, H), jnp.float32),
      ],
  )(accp, zh, dinv, hprev, b, g, be, w)


def _tc_final_body(accp_ref, zh_ref, dinv_ref, hprev_ref, b_ref, g_ref,
                   be_ref, a1w_ref, a1b_ref, a2w_ref, a2b_ref, p1w_ref,
                   p1b_ref, p2w_ref, p2b_ref, batch_ref, out_ref):
  y, _ = _layer_math(accp_ref, zh_ref, dinv_ref, b_ref, g_ref, be_ref)
  h = y + hprev_ref[:, :]
  t = jnp.tanh(jnp.dot(h, a1w_ref[:, :], preferred_element_type=jnp.float32)
               + a1b_ref[:, :])
  s = jnp.sum(t * a2w_ref[:, :], axis=1, keepdims=True) + a2b_ref[:, :]
  e = jnp.exp(s - jnp.max(s))
  attn = e / jnp.sum(e)
  hw = h * attn
  bt = batch_ref[:, :]
  gid = lax.broadcasted_iota(jnp.int32, (G, N), 0)
  onehot_t = (gid == bt).astype(jnp.float32)
  sums = jnp.dot(onehot_t, hw, preferred_element_type=jnp.float32)
  cnt = jnp.sum(onehot_t, axis=1, keepdims=True)
  pooled = sums / jnp.maximum(cnt, 1.0)
  t1 = jnp.maximum(
      jnp.dot(pooled, p1w_ref[:, :], preferred_element_type=jnp.float32)
      + p1b_ref[:, :], 0.0)
  out_ref[:, :] = jnp.dot(
      t1, p2w_ref[:, :], preferred_element_type=jnp.float32) + p2b_ref[:, :]


def _tc_final(accp, zh, dinv, hprev, b, g, be,
              a1w, a1b, a2w, a2b, p1w, p1b, p2w, p2b, batch2d):
  return pl.pallas_call(
      _tc_final_body,
      out_shape=jax.ShapeDtypeStruct((G, H), jnp.float32),
  )(accp, zh, dinv, hprev, b, g, be,
    a1w, a1b, a2w, a2b, p1w, p1b, p2w, p2b, batch2d)


def kernel(x, edge_index, batch, W0, b0, W1, b1, W2, b2, g0, be0, g1, be1,
           g2, be2, A1w, A1b, A2w, A2b, P1w, P1b, P2w, P2b):
  num_edges = edge_index.shape[1]
  chunks = _num_chunks(num_edges)
  epad = NW * chunks * CHUNK
  pad = epad - num_edges
  src3 = jnp.concatenate(
      [edge_index[0], jnp.zeros((pad,), jnp.int32)]).reshape(NW, chunks, CHUNK)
  dst3 = jnp.concatenate(
      [edge_index[1], jnp.full((pad,), N, jnp.int32)]).reshape(NW, chunks, CHUNK)

  degp = _deg_call(dst3)
  degt = degp.T

  b0r, b1r, b2r = (v.reshape(1, H) for v in (b0, b1, b2))
  g0r, g1r, g2r = (v.reshape(1, H) for v in (g0, g1, g2))
  be0r, be1r, be2r = (v.reshape(1, H) for v in (be0, be1, be2))

  zh0, dinv = _tc_first(degt, x, W0)
  acc1 = _agg_call(zh0, src3, dst3)
  h1, zh1 = _tc_mid(False, acc1, zh0, dinv, x, b0r, g0r, be0r, W1)
  acc2 = _agg_call(zh1, src3, dst3)
  h2, zh2 = _tc_mid(True, acc2, zh1, dinv, h1, b1r, g1r, be1r, W2)
  acc3 = _agg_call(zh2, src3, dst3)
  out = _tc_final(acc3, zh2, dinv, h2, b2r, g2r, be2r,
                  A1w, A1b.reshape(1, H), A2w.reshape(1, H),
                  A2b.reshape(1, 1), P1w, P1b.reshape(1, H),
                  P2w, P2b.reshape(1, H), batch.reshape(1, N))
  return out


# speedup vs baseline: 9.0910x; 1.1592x over previous
import functools

import jax
import jax.numpy as jnp
from jax import lax
from jax.experimental import pallas as pl
from jax.experimental.pallas import tpu as pltpu
from jax.experimental.pallas import tpu_sc as plsc

N = 10000
H = 128
G = 64
NC = 2
NS = 16
NW = NC * NS
LANES = 16
CHUNK = 64
SG = 16
NBUF = 4
NPAD = 10112
ROWS_PER_TILE = NPAD // NS
ZB = CHUNK

_MESH = plsc.VectorSubcoreMesh(
    core_axis_name="c", subcore_axis_name="s", num_cores=NC, num_subcores=NS)


def _num_chunks(num_edges):
  c = -(-num_edges // (NW * CHUNK))
  return -(-c // (2 * SG)) * (2 * SG)


def _deg_body(chunks, dst_hbm, out_hbm, didx, degv):
  wid = lax.axis_index("s") * NC + lax.axis_index("c")
  pltpu.sync_copy(dst_hbm.at[wid], didx)

  zeros16 = jnp.zeros((LANES,), jnp.float32)

  def zero_body(i, _):
    degv[pl.ds(i * LANES, LANES)] = zeros16
    return 0

  lax.fori_loop(0, NPAD // LANES, zero_body, 0)

  ones16 = jnp.ones((LANES,), jnp.float32)

  def edge_body(j, _):
    for k in range(CHUNK // LANES):
      idx = didx[j, pl.ds(k * LANES, LANES)]
      plsc.addupdate_scatter(degv, [idx], ones16)
    return 0

  lax.fori_loop(0, chunks, edge_body, 0)
  pltpu.sync_copy(degv, out_hbm.at[wid])


def _deg_call(dst3):
  chunks = dst3.shape[1]
  fn = pl.kernel(
      functools.partial(_deg_body, chunks),
      out_type=jax.ShapeDtypeStruct((NW, NPAD), jnp.float32),
      mesh=_MESH,
      compiler_params=pltpu.CompilerParams(needs_layout_passes=False),
      scratch_types=[
          pltpu.VMEM((chunks, CHUNK), jnp.int32),
          pltpu.VMEM((NPAD,), jnp.float32),
      ],
  )
  return fn(dst3)


def _agg_body(chunks, zh_hbm, src_hbm, dst_hbm, out_hbm,
              sidx0, didx0, sidx1, didx1, dbuf,
              msg0, msg1, msg2, msg3, acc,
              gs0, gs1, gs2, gs3, ss0, ss1, ss2, ss3, isem):
  cid = lax.axis_index("c")
  sid = lax.axis_index("s")
  wid = sid * NC + cid
  ngroups = chunks // SG
  msg = [msg0, msg1, msg2, msg3]
  gsem = [gs0, gs1, gs2, gs3]
  ssem = [ss0, ss1, ss2, ss3]

  zeros16 = jnp.zeros((LANES,), jnp.float32)

  def zb_body(i, _):
    for c in range(H // LANES):
      msg0[i, pl.ds(c * LANES, LANES)] = zeros16
    return 0

  lax.fori_loop(0, ZB, zb_body, 0)

  dummy16 = jnp.full((LANES,), N, jnp.int32)
  for c in range(CHUNK // LANES):
    dbuf[0, pl.ds(c * LANES, LANES)] = dummy16

  base = sid * ROWS_PER_TILE
  nfull = ROWS_PER_TILE // ZB
  rem = ROWS_PER_TILE % ZB
  for k in range(nfull):
    pltpu.sync_copy(msg0, acc.at[pl.ds(base + k * ZB, ZB)])
  if rem:
    pltpu.sync_copy(msg0.at[pl.ds(0, rem)],
                    acc.at[pl.ds(base + nfull * ZB, rem)])

  for p in range(NBUF):
    pltpu.async_copy(msg[p], acc.at[dbuf.at[0]], ssem[p], add=True)
  pltpu.async_copy(src_hbm.at[wid, pl.ds(0, SG)], sidx0, isem)
  pltpu.async_copy(dst_hbm.at[wid, pl.ds(0, SG)], didx0, isem)
  plsc.subcore_barrier()

  def do_group(q, s_cur, d_cur, s_nxt, d_nxt):
    pltpu.make_async_copy(
        src_hbm.at[wid, pl.ds(q * SG, SG)], s_cur, isem).wait()
    pltpu.make_async_copy(
        dst_hbm.at[wid, pl.ds(q * SG, SG)], d_cur, isem).wait()
    qn = jnp.where(q + 1 < ngroups, q + 1, 0)
    for k in range(SG):
      p = k % NBUF
      pltpu.make_async_copy(msg[p], acc.at[dbuf.at[0]], ssem[p]).wait()
      pltpu.async_copy(zh_hbm.at[s_cur.at[k]], msg[p], gsem[p])
      if k == 4:
        pltpu.async_copy(src_hbm.at[wid, pl.ds(qn * SG, SG)], s_nxt, isem)
        pltpu.async_copy(dst_hbm.at[wid, pl.ds(qn * SG, SG)], d_nxt, isem)
      if k >= 2:
        p2 = (k - 2) % NBUF
        pltpu.make_async_copy(
            zh_hbm.at[s_cur.at[k - 2]], msg[p2], gsem[p2]).wait()
        pltpu.async_copy(msg[p2], acc.at[d_cur.at[k - 2]], ssem[p2],
                         add=True)
    for kk in (SG - 2, SG - 1):
      p2 = kk % NBUF
      pltpu.make_async_copy(
          zh_hbm.at[s_cur.at[kk]], msg[p2], gsem[p2]).wait()
      pltpu.async_copy(msg[p2], acc.at[d_cur.at[kk]], ssem[p2],
                       add=True)

  def pair_body(i, _):
    do_group(2 * i, sidx0, didx0, sidx1, didx1)
    do_group(2 * i + 1, sidx1, didx1, sidx0, didx0)
    return 0

  lax.fori_loop(0, ngroups // 2, pair_body, 0)

  pltpu.make_async_copy(src_hbm.at[wid, pl.ds(0, SG)], sidx0, isem).wait()
  pltpu.make_async_copy(dst_hbm.at[wid, pl.ds(0, SG)], didx0, isem).wait()
  for p in range(NBUF):
    pltpu.make_async_copy(msg[p], acc.at[dbuf.at[0]], ssem[p]).wait()
  plsc.subcore_barrier()
  pltpu.sync_copy(acc.at[pl.ds(base, ROWS_PER_TILE)],
                  out_hbm.at[cid, pl.ds(base, ROWS_PER_TILE)])


def _agg_call(zh, src3, dst3):
  chunks = src3.shape[1]
  fn = pl.kernel(
      functools.partial(_agg_body, chunks),
      out_type=jax.ShapeDtypeStruct((NC, NPAD, H), jnp.float32),
      mesh=_MESH,
      compiler_params=pltpu.CompilerParams(needs_layout_passes=False),
      scratch_types=(
          [pltpu.VMEM((SG, CHUNK), jnp.int32)] * 4
          + [pltpu.VMEM((1, CHUNK), jnp.int32)]
          + [pltpu.VMEM((CHUNK, H), jnp.float32)] * 4
          + [pltpu.VMEM_SHARED((NPAD, H), jnp.float32)]
          + [pltpu.SemaphoreType.DMA] * 9
      ),
  )
  return fn(zh, src3, dst3)


def _tc_first_body(degt_ref, x_ref, w_ref, zh_ref, dinv_ref):
  deg = jnp.sum(degt_ref[:, :], axis=1, keepdims=True) + 1.0
  dinv = lax.rsqrt(deg)[:N]
  z = jnp.dot(x_ref[:, :], w_ref[:, :], preferred_element_type=jnp.float32)
  zh_ref[:, :] = dinv * z
  dinv_ref[:, :] = dinv


def _tc_first(degt, x, w0):
  return pl.pallas_call(
      _tc_first_body,
      out_shape=[
          jax.ShapeDtypeStruct((N, H), jnp.float32),
          jax.ShapeDtypeStruct((N, 1), jnp.float32),
      ],
  )(degt, x, w0)


def _layer_math(accp_ref, zh_ref, dinv_ref, b_ref, g_ref, be_ref):
  acc = accp_ref[0, :N, :] + accp_ref[1, :N, :]
  dinv = dinv_ref[:, :]
  u = dinv * (acc + zh_ref[:, :]) + b_ref[:, :]
  m = jnp.mean(u, axis=0, keepdims=True)
  v = jnp.mean((u - m) * (u - m), axis=0, keepdims=True)
  y = g_ref[:, :] * (u - m) / jnp.sqrt(v + 1e-5) + be_ref[:, :]
  return jnp.maximum(y, 0.0), dinv


def _tc_mid_body(residual, accp_ref, zh_ref, dinv_ref, hprev_ref,
                 b_ref, g_ref, be_ref, w_ref, h_ref, zhn_ref):
  y, dinv = _layer_math(accp_ref, zh_ref, dinv_ref, b_ref, g_ref, be_ref)
  if residual:
    y = y + hprev_ref[:, :]
  h_ref[:, :] = y
  zhn_ref[:, :] = dinv * jnp.dot(
      y, w_ref[:, :], preferred_element_type=jnp.float32)


def _tc_mid(residual, accp, zh, dinv, hprev, b, g, be, w):
  return pl.pallas_call(
      functools.partial(_tc_mid_body, residual),
      out_shape=[
          jax.ShapeDtypeStruct((N, H), jnp.float32),
          jax.ShapeDtypeStruct((N, H), jnp.float32),
      ],
  )(accp, zh, dinv, hprev, b, g, be, w)


def _tc_final_body(accp_ref, zh_ref, dinv_ref, hprev_ref, b_ref, g_ref,
                   be_ref, a1w_ref, a1b_ref, a2w_ref, a2b_ref, p1w_ref,
                   p1b_ref, p2w_ref, p2b_ref, batch_ref, out_ref):
  y, _ = _layer_math(accp_ref, zh_ref, dinv_ref, b_ref, g_ref, be_ref)
  h = y + hprev_ref[:, :]
  t = jnp.tanh(jnp.dot(h, a1w_ref[:, :], preferred_element_type=jnp.float32)
               + a1b_ref[:, :])
  s = jnp.sum(t * a2w_ref[:, :], axis=1, keepdims=True) + a2b_ref[:, :]
  e = jnp.exp(s - jnp.max(s))
  attn = e / jnp.sum(e)
  hw = h * attn
  bt = batch_ref[:, :]
  gid = lax.broadcasted_iota(jnp.int32, (G, N), 0)
  onehot_t = (gid == bt).astype(jnp.float32)
  sums = jnp.dot(onehot_t, hw, preferred_element_type=jnp.float32)
  cnt = jnp.sum(onehot_t, axis=1, keepdims=True)
  pooled = sums / jnp.maximum(cnt, 1.0)
  t1 = jnp.maximum(
      jnp.dot(pooled, p1w_ref[:, :], preferred_element_type=jnp.float32)
      + p1b_ref[:, :], 0.0)
  out_ref[:, :] = jnp.dot(
      t1, p2w_ref[:, :], preferred_element_type=jnp.float32) + p2b_ref[:, :]


def _tc_final(accp, zh, dinv, hprev, b, g, be,
              a1w, a1b, a2w, a2b, p1w, p1b, p2w, p2b, batch2d):
  return pl.pallas_call(
      _tc_final_body,
      out_shape=jax.ShapeDtypeStruct((G, H), jnp.float32),
  )(accp, zh, dinv, hprev, b, g, be,
    a1w, a1b, a2w, a2b, p1w, p1b, p2w, p2b, batch2d)


def kernel(x, edge_index, batch, W0, b0, W1, b1, W2, b2, g0, be0, g1, be1,
           g2, be2, A1w, A1b, A2w, A2b, P1w, P1b, P2w, P2b):
  num_edges = edge_index.shape[1]
  chunks = _num_chunks(num_edges)
  epad = NW * chunks * CHUNK
  pad = epad - num_edges
  src3 = jnp.concatenate(
      [edge_index[0], jnp.zeros((pad,), jnp.int32)]).reshape(NW, chunks, CHUNK)
  dst3 = jnp.concatenate(
      [edge_index[1], jnp.full((pad,), N, jnp.int32)]).reshape(NW, chunks, CHUNK)

  degp = _deg_call(dst3)
  degt = degp.T

  b0r, b1r, b2r = (v.reshape(1, H) for v in (b0, b1, b2))
  g0r, g1r, g2r = (v.reshape(1, H) for v in (g0, g1, g2))
  be0r, be1r, be2r = (v.reshape(1, H) for v in (be0, be1, be2))

  zh0, dinv = _tc_first(degt, x, W0)
  acc1 = _agg_call(zh0, src3, dst3)
  h1, zh1 = _tc_mid(False, acc1, zh0, dinv, x, b0r, g0r, be0r, W1)
  acc2 = _agg_call(zh1, src3, dst3)
  h2, zh2 = _tc_mid(True, acc2, zh1, dinv, h1, b1r, g1r, be1r, W2)
  acc3 = _agg_call(zh2, src3, dst3)
  out = _tc_final(acc3, zh2, dinv, h2, b2r, g2r, be2r,
                  A1w, A1b.reshape(1, H), A2w.reshape(1, H),
                  A2b.reshape(1, 1), P1w, P1b.reshape(1, H),
                  P2w, P2b.reshape(1, H), batch.reshape(1, N))
  return out


# speedup vs baseline: 9.4789x; 1.0427x over previous
import functools

import jax
import jax.numpy as jnp
from jax import lax
from jax.experimental import pallas as pl
from jax.experimental.pallas import tpu as pltpu
from jax.experimental.pallas import tpu_sc as plsc

N = 10000
H = 128
G = 64
NC = 2
NS = 16
NW = NC * NS
LANES = 16
CHUNK = 64
SG = 16
NBUF = 4
NPAD = 10112
ROWS_PER_TILE = NPAD // NS
ZB = CHUNK

_MESH = plsc.VectorSubcoreMesh(
    core_axis_name="c", subcore_axis_name="s", num_cores=NC, num_subcores=NS)


def _split_chunks(num_edges, share0=0.82):
  total = -(-num_edges // CHUNK)
  quantum = 2 * SG * NS
  t0 = int(total * share0)
  k0 = max(2 * SG, (t0 // quantum) * (2 * SG))
  rem = max(0, total - k0 * NS)
  k1 = max(2 * SG, -(-rem // quantum) * (2 * SG))
  return k0, k1


def _deg_body(cpt, dst_hbm, out_hbm, didx, degv):
  wid = lax.axis_index("s") * NC + lax.axis_index("c")
  pltpu.sync_copy(dst_hbm.at[pl.ds(wid * cpt, cpt)], didx)

  zeros16 = jnp.zeros((LANES,), jnp.float32)

  def zero_body(i, _):
    degv[pl.ds(i * LANES, LANES)] = zeros16
    return 0

  lax.fori_loop(0, NPAD // LANES, zero_body, 0)

  ones16 = jnp.ones((LANES,), jnp.float32)

  def edge_body(j, _):
    for k in range(CHUNK // LANES):
      idx = didx[j, pl.ds(k * LANES, LANES)]
      plsc.addupdate_scatter(degv, [idx], ones16)
    return 0

  lax.fori_loop(0, cpt, edge_body, 0)
  pltpu.sync_copy(degv, out_hbm.at[wid])


def _deg_call(dst2):
  cpt = dst2.shape[0] // NW
  fn = pl.kernel(
      functools.partial(_deg_body, cpt),
      out_type=jax.ShapeDtypeStruct((NW, NPAD), jnp.float32),
      mesh=_MESH,
      compiler_params=pltpu.CompilerParams(needs_layout_passes=False),
      scratch_types=[
          pltpu.VMEM((cpt, CHUNK), jnp.int32),
          pltpu.VMEM((NPAD,), jnp.float32),
      ],
  )
  return fn(dst2)


def _agg_body(k0, k1, zh_hbm, src_hbm, dst_hbm, out_hbm,
              sidx0, didx0, sidx1, didx1, dbuf,
              msg0, msg1, msg2, msg3, acc,
              gs0, gs1, gs2, gs3, ss0, ss1, ss2, ss3, isem):
  cid = lax.axis_index("c")
  sid = lax.axis_index("s")
  base_chunk = jnp.where(cid == 0, sid * k0, NS * k0 + sid * k1)
  ngroups = jnp.where(cid == 0, k0 // SG, k1 // SG)
  msg = [msg0, msg1, msg2, msg3]
  gsem = [gs0, gs1, gs2, gs3]
  ssem = [ss0, ss1, ss2, ss3]

  zeros16 = jnp.zeros((LANES,), jnp.float32)

  def zb_body(i, _):
    for c in range(H // LANES):
      msg0[i, pl.ds(c * LANES, LANES)] = zeros16
    return 0

  lax.fori_loop(0, ZB, zb_body, 0)

  dummy16 = jnp.full((LANES,), N, jnp.int32)
  for c in range(CHUNK // LANES):
    dbuf[0, pl.ds(c * LANES, LANES)] = dummy16

  base = sid * ROWS_PER_TILE
  nfull = ROWS_PER_TILE // ZB
  rem = ROWS_PER_TILE % ZB
  for k in range(nfull):
    pltpu.sync_copy(msg0, acc.at[pl.ds(base + k * ZB, ZB)])
  if rem:
    pltpu.sync_copy(msg0.at[pl.ds(0, rem)],
                    acc.at[pl.ds(base + nfull * ZB, rem)])

  for p in range(NBUF):
    pltpu.async_copy(msg[p], acc.at[dbuf.at[0]], ssem[p], add=True)
  pltpu.async_copy(src_hbm.at[pl.ds(base_chunk, SG)], sidx0, isem)
  pltpu.async_copy(dst_hbm.at[pl.ds(base_chunk, SG)], didx0, isem)
  plsc.subcore_barrier()

  def do_group(q, s_cur, d_cur, s_nxt, d_nxt):
    row = base_chunk + q * SG
    pltpu.make_async_copy(src_hbm.at[pl.ds(row, SG)], s_cur, isem).wait()
    pltpu.make_async_copy(dst_hbm.at[pl.ds(row, SG)], d_cur, isem).wait()
    rown = base_chunk + jnp.where(q + 1 < ngroups, (q + 1) * SG, 0)
    for k in range(SG):
      p = k % NBUF
      pltpu.make_async_copy(msg[p], acc.at[dbuf.at[0]], ssem[p]).wait()
      pltpu.async_copy(zh_hbm.at[s_cur.at[k]], msg[p], gsem[p])
      if k == 4:
        pltpu.async_copy(src_hbm.at[pl.ds(rown, SG)], s_nxt, isem)
        pltpu.async_copy(dst_hbm.at[pl.ds(rown, SG)], d_nxt, isem)
      if k >= 2:
        p2 = (k - 2) % NBUF
        pltpu.make_async_copy(
            zh_hbm.at[s_cur.at[k - 2]], msg[p2], gsem[p2]).wait()
        pltpu.async_copy(msg[p2], acc.at[d_cur.at[k - 2]], ssem[p2],
                         add=True)
    for kk in (SG - 2, SG - 1):
      p2 = kk % NBUF
      pltpu.make_async_copy(
          zh_hbm.at[s_cur.at[kk]], msg[p2], gsem[p2]).wait()
      pltpu.async_copy(msg[p2], acc.at[d_cur.at[kk]], ssem[p2],
                       add=True)

  def pair_body(i, _):
    do_group(2 * i, sidx0, didx0, sidx1, didx1)
    do_group(2 * i + 1, sidx1, didx1, sidx0, didx0)
    return 0

  lax.fori_loop(0, ngroups // 2, pair_body, 0)

  pltpu.make_async_copy(
      src_hbm.at[pl.ds(base_chunk, SG)], sidx0, isem).wait()
  pltpu.make_async_copy(
      dst_hbm.at[pl.ds(base_chunk, SG)], didx0, isem).wait()
  for p in range(NBUF):
    pltpu.make_async_copy(msg[p], acc.at[dbuf.at[0]], ssem[p]).wait()
  plsc.subcore_barrier()
  pltpu.sync_copy(acc.at[pl.ds(base, ROWS_PER_TILE)],
                  out_hbm.at[cid, pl.ds(base, ROWS_PER_TILE)])


def _agg_call(zh, src2, dst2, k0, k1):
  fn = pl.kernel(
      functools.partial(_agg_body, k0, k1),
      out_type=jax.ShapeDtypeStruct((NC, NPAD, H), jnp.float32),
      mesh=_MESH,
      compiler_params=pltpu.CompilerParams(needs_layout_passes=False),
      scratch_types=(
          [pltpu.VMEM((SG, CHUNK), jnp.int32)] * 4
          + [pltpu.VMEM((1, CHUNK), jnp.int32)]
          + [pltpu.VMEM((CHUNK, H), jnp.float32)] * 4
          + [pltpu.VMEM_SHARED((NPAD, H), jnp.float32)]
          + [pltpu.SemaphoreType.DMA] * 9
      ),
  )
  return fn(zh, src2, dst2)


def _tc_first_body(degt_ref, x_ref, w_ref, zh_ref, dinv_ref):
  deg = jnp.sum(degt_ref[:, :], axis=1, keepdims=True) + 1.0
  dinv = lax.rsqrt(deg)[:N]
  z = jnp.dot(x_ref[:, :], w_ref[:, :], preferred_element_type=jnp.float32)
  zh_ref[:, :] = dinv * z
  dinv_ref[:, :] = dinv


def _tc_first(degt, x, w0):
  return pl.pallas_call(
      _tc_first_body,
      out_shape=[
          jax.ShapeDtypeStruct((N, H), jnp.float32),
          jax.ShapeDtypeStruct((N, 1), jnp.float32),
      ],
  )(degt, x, w0)


def _layer_math(accp_ref, zh_ref, dinv_ref, b_ref, g_ref, be_ref):
  acc = accp_ref[0, :N, :] + accp_ref[1, :N, :]
  dinv = dinv_ref[:, :]
  u = dinv * (acc + zh_ref[:, :]) + b_ref[:, :]
  m = jnp.mean(u, axis=0, keepdims=True)
  v = jnp.mean((u - m) * (u - m), axis=0, keepdims=True)
  y = g_ref[:, :] * (u - m) / jnp.sqrt(v + 1e-5) + be_ref[:, :]
  return jnp.maximum(y, 0.0), dinv


def _tc_mid_body(residual, accp_ref, zh_ref, dinv_ref, hprev_ref,
                 b_ref, g_ref, be_ref, w_ref, h_ref, zhn_ref):
  y, dinv = _layer_math(accp_ref, zh_ref, dinv_ref, b_ref, g_ref, be_ref)
  if residual:
    y = y + hprev_ref[:, :]
  h_ref[:, :] = y
  zhn_ref[:, :] = dinv * jnp.dot(
      y, w_ref[:, :], preferred_element_type=jnp.float32)


def _tc_mid(residual, accp, zh, dinv, hprev, b, g, be, w):
  return pl.pallas_call(
      functools.partial(_tc_mid_body, residual),
      out_shape=[
          jax.ShapeDtypeStruct((N, H), jnp.float32),
          jax.ShapeDtypeStruct((N, H), jnp.float32),
      ],
  )(accp, zh, dinv, hprev, b, g, be, w)


def _tc_final_body(accp_ref, zh_ref, dinv_ref, hprev_ref, b_ref, g_ref,
                   be_ref, a1w_ref, a1b_ref, a2w_ref, a2b_ref, p1w_ref,
                   p1b_ref, p2w_ref, p2b_ref, batch_ref, out_ref):
  y, _ = _layer_math(accp_ref, zh_ref, dinv_ref, b_ref, g_ref, be_ref)
  h = y + hprev_ref[:, :]
  t = jnp.tanh(jnp.dot(h, a1w_ref[:, :], preferred_element_type=jnp.float32)
               + a1b_ref[:, :])
  s = jnp.sum(t * a2w_ref[:, :], axis=1, keepdims=True) + a2b_ref[:, :]
  e = jnp.exp(s - jnp.max(s))
  attn = e / jnp.sum(e)
  hw = h * attn
  bt = batch_ref[:, :]
  gid = lax.broadcasted_iota(jnp.int32, (G, N), 0)
  onehot_t = (gid == bt).astype(jnp.float32)
  sums = jnp.dot(onehot_t, hw, preferred_element_type=jnp.float32)
  cnt = jnp.sum(onehot_t, axis=1, keepdims=True)
  pooled = sums / jnp.maximum(cnt, 1.0)
  t1 = jnp.maximum(
      jnp.dot(pooled, p1w_ref[:, :], preferred_element_type=jnp.float32)
      + p1b_ref[:, :], 0.0)
  out_ref[:, :] = jnp.dot(
      t1, p2w_ref[:, :], preferred_element_type=jnp.float32) + p2b_ref[:, :]


def _tc_final(accp, zh, dinv, hprev, b, g, be,
              a1w, a1b, a2w, a2b, p1w, p1b, p2w, p2b, batch2d):
  return pl.pallas_call(
      _tc_final_body,
      out_shape=jax.ShapeDtypeStruct((G, H), jnp.float32),
  )(accp, zh, dinv, hprev, b, g, be,
    a1w, a1b, a2w, a2b, p1w, p1b, p2w, p2b, batch2d)


def kernel(x, edge_index, batch, W0, b0, W1, b1, W2, b2, g0, be0, g1, be1,
           g2, be2, A1w, A1b, A2w, A2b, P1w, P1b, P2w, P2b):
  num_edges = edge_index.shape[1]
  k0, k1 = _split_chunks(num_edges)
  totc = NS * (k0 + k1)
  pad = totc * CHUNK - num_edges
  src2 = jnp.concatenate(
      [edge_index[0], jnp.zeros((pad,), jnp.int32)]).reshape(totc, CHUNK)
  dst2 = jnp.concatenate(
      [edge_index[1], jnp.full((pad,), N, jnp.int32)]).reshape(totc, CHUNK)

  degp = _deg_call(dst2)
  degt = degp.T

  b0r, b1r, b2r = (v.reshape(1, H) for v in (b0, b1, b2))
  g0r, g1r, g2r = (v.reshape(1, H) for v in (g0, g1, g2))
  be0r, be1r, be2r = (v.reshape(1, H) for v in (be0, be1, be2))

  zh0, dinv = _tc_first(degt, x, W0)
  acc1 = _agg_call(zh0, src2, dst2, k0, k1)
  h1, zh1 = _tc_mid(False, acc1, zh0, dinv, x, b0r, g0r, be0r, W1)
  acc2 = _agg_call(zh1, src2, dst2, k0, k1)
  h2, zh2 = _tc_mid(True, acc2, zh1, dinv, h1, b1r, g1r, be1r, W2)
  acc3 = _agg_call(zh2, src2, dst2, k0, k1)
  out = _tc_final(acc3, zh2, dinv, h2, b2r, g2r, be2r,
                  A1w, A1b.reshape(1, H), A2w.reshape(1, H),
                  A2b.reshape(1, 1), P1w, P1b.reshape(1, H),
                  P2w, P2b.reshape(1, H), batch.reshape(1, N))
  return out
